# Initial kernel scaffold; baseline (speedup 1.0000x reference)
import functools

import jax
import jax.numpy as jnp
from jax import lax
from jax.experimental import pallas as pl
from jax.experimental.pallas import tpu as pltpu
from jax.experimental.pallas import tpu_sc as plsc

Q = 1024
D = 768
K = 100000
TK = 5
KB = 2048
NB = (K + KB - 1) // KB

_NC = 2
_NS = 16
_NW = _NC * _NS
_RPW = (Q * TK) // _NW
_CH = _RPW // 2


def _topk_body(q_ref, k_ref, vals_ref, idx_ref, qinv_ref, rv_ref, ri_ref):
    b = pl.program_id(0)

    @pl.when(b == 0)
    def _init():
        q = q_ref[...]
        qs = jnp.sum(q * q, axis=1, keepdims=True)
        qinv_ref[...] = 1.0 / (jnp.sqrt(qs) + 1e-12)
        rv_ref[...] = jnp.full((Q, TK), -jnp.inf, jnp.float32)
        ri_ref[...] = jnp.zeros((Q, TK), jnp.int32)

    kb = k_ref[...]
    ksq = jnp.sum(kb * kb, axis=1)
    kinv = 1.0 / (jnp.sqrt(ksq) + 1e-12)
    s = lax.dot_general(
        q_ref[...], kb, (((1,), (1,)), ((), ())),
        preferred_element_type=jnp.float32,
        precision=lax.Precision.HIGHEST,
    )
    s = s * kinv[None, :]
    col = lax.broadcasted_iota(jnp.int32, (Q, KB), 1)
    base = b * KB
    s = jnp.where(col + base < K, s, -jnp.inf)

    lv, li = [], []
    for _ in range(TK):
        m = jnp.max(s, axis=1)
        am = jnp.min(jnp.where(s == m[:, None], col, KB), axis=1)
        lv.append(m[:, None])
        li.append(am[:, None] + base)
        s = jnp.where(col == am[:, None], -jnp.inf, s)

    cv = jnp.concatenate([rv_ref[...]] + lv, axis=1)
    ci = jnp.concatenate([ri_ref[...]] + li, axis=1)
    nv, ni = [], []
    for _ in range(TK):
        m = jnp.max(cv, axis=1)
        pick = jnp.min(jnp.where(cv == m[:, None], ci, K), axis=1)
        nv.append(m[:, None])
        ni.append(pick[:, None])
        cv = jnp.where(ci == pick[:, None], -jnp.inf, cv)
    rv_ref[...] = jnp.concatenate(nv, axis=1)
    ri_ref[...] = jnp.concatenate(ni, axis=1)

    @pl.when(b == NB - 1)
    def _fin():
        vals_ref[...] = rv_ref[...] * qinv_ref[...]
        idx_ref[...] = ri_ref[...]


_topk_call = pl.pallas_call(
    _topk_body,
    grid=(NB,),
    in_specs=[
        pl.BlockSpec((Q, D), lambda b: (0, 0)),
        pl.BlockSpec((KB, D), lambda b: (b, 0)),
    ],
    out_specs=[
        pl.BlockSpec((Q, TK), lambda b: (0, 0)),
        pl.BlockSpec((Q, TK), lambda b: (0, 0)),
    ],
    out_shape=[
        jax.ShapeDtypeStruct((Q, TK), jnp.float32),
        jax.ShapeDtypeStruct((Q, TK), jnp.int32),
    ],
    scratch_shapes=[
        pltpu.VMEM((Q, 1), jnp.float32),
        pltpu.VMEM((Q, TK), jnp.float32),
        pltpu.VMEM((Q, TK), jnp.int32),
    ],
    compiler_params=pltpu.CompilerParams(dimension_semantics=("arbitrary",)),
)


def _gather_body(idx_hbm, tab_hbm, out_hbm, idx_v, rows_v, sem):
    wid = lax.axis_index("s") * _NC + lax.axis_index("c")
    base = wid * _RPW
    pltpu.sync_copy(idx_hbm.at[pl.ds(base, _RPW)], idx_v)
    c0 = pltpu.async_copy(
        tab_hbm.at[idx_v.at[pl.ds(0, _CH)]], rows_v.at[pl.ds(0, _CH)], sem)
    c1 = pltpu.async_copy(
        tab_hbm.at[idx_v.at[pl.ds(_CH, _CH)]], rows_v.at[pl.ds(_CH, _CH)], sem)
    c0.wait()
    c1.wait()
    pltpu.sync_copy(rows_v, out_hbm.at[pl.ds(base, _RPW)])


_gather_call = pl.kernel(
    _gather_body,
    out_type=jax.ShapeDtypeStruct((Q * TK, D), jnp.float32),
    mesh=plsc.VectorSubcoreMesh(core_axis_name="c", subcore_axis_name="s"),
    scratch_types=[
        pltpu.VMEM((_RPW,), jnp.int32),
        pltpu.VMEM((_RPW, D), jnp.float32),
        pltpu.SemaphoreType.DMA,
    ],
)


def _evnorm_body(e_ref, o_ref):
    e = e_ref[...]
    sq = jnp.sum(e * e, axis=1, keepdims=True)
    o_ref[...] = e / (jnp.sqrt(sq) + 1e-12)


_evnorm_call = pl.pallas_call(
    _evnorm_body,
    grid=(8,),
    in_specs=[pl.BlockSpec((Q * TK // 8, D), lambda i: (i, 0))],
    out_specs=pl.BlockSpec((Q * TK // 8, D), lambda i: (i, 0)),
    out_shape=jax.ShapeDtypeStruct((Q * TK, D), jnp.float32),
)


def kernel(queries, keys):
    top_vals, top_idx = _topk_call(queries, keys)
    flat_idx = top_idx.reshape(Q * TK)
    ev_raw = _gather_call(flat_idx, keys)
    evidence = _evnorm_call(ev_raw).reshape(Q, TK, D)
    return top_vals, top_idx, evidence


# speedup vs baseline: 2.2815x; 2.2815x over previous
import functools

import jax
import jax.numpy as jnp
from jax import lax
from jax.experimental import pallas as pl
from jax.experimental.pallas import tpu as pltpu
from jax.experimental.pallas import tpu_sc as plsc

Q = 1024
D = 768
K = 100000
TK = 5
KB = 2048
NB = (K + KB - 1) // KB

_NC = 2
_NS = 16
_NW = _NC * _NS
_RPW = (Q * TK) // _NW
_CH = _RPW // 2


def _topk_body(q_ref, k_ref, vals_ref, idx_ref, qbf_ref, rv_ref, ri_ref):
    b = pl.program_id(0)

    @pl.when(b == 0)
    def _init():
        q = q_ref[...]
        qs = jnp.sum(q * q, axis=1, keepdims=True)
        qn = q * (1.0 / (jnp.sqrt(qs) + 1e-12))
        qbf_ref[...] = qn.astype(jnp.bfloat16)
        rv_ref[...] = jnp.full((Q, TK), -jnp.inf, jnp.float32)
        ri_ref[...] = jnp.zeros((Q, TK), jnp.int32)

    kb = k_ref[...]
    ksq = jnp.sum(kb * kb, axis=1)
    kinv = 1.0 / (jnp.sqrt(ksq) + 1e-12)
    knb = (kb * kinv[:, None]).astype(jnp.bfloat16)
    s = lax.dot_general(
        qbf_ref[...], knb, (((1,), (1,)), ((), ())),
        preferred_element_type=jnp.float32,
    )
    col = lax.broadcasted_iota(jnp.int32, (Q, KB), 1)
    base = b * KB
    s = jnp.where(col + base < K, s, -jnp.inf)

    lv, li = [], []
    for _ in range(TK):
        m = jnp.max(s, axis=1)
        am = jnp.min(jnp.where(s == m[:, None], col, KB), axis=1)
        lv.append(m[:, None])
        li.append(am[:, None] + base)
        s = jnp.where(col == am[:, None], -jnp.inf, s)

    cv = jnp.concatenate([rv_ref[...]] + lv, axis=1)
    ci = jnp.concatenate([ri_ref[...]] + li, axis=1)
    nv, ni = [], []
    for _ in range(TK):
        m = jnp.max(cv, axis=1)
        pick = jnp.min(jnp.where(cv == m[:, None], ci, K), axis=1)
        nv.append(m[:, None])
        ni.append(pick[:, None])
        cv = jnp.where(ci == pick[:, None], -jnp.inf, cv)
    rv_ref[...] = jnp.concatenate(nv, axis=1)
    ri_ref[...] = jnp.concatenate(ni, axis=1)

    @pl.when(b == NB - 1)
    def _fin():
        vals_ref[...] = rv_ref[...]
        idx_ref[...] = ri_ref[...]


_topk_call = pl.pallas_call(
    _topk_body,
    grid=(NB,),
    in_specs=[
        pl.BlockSpec((Q, D), lambda b: (0, 0)),
        pl.BlockSpec((KB, D), lambda b: (b, 0)),
    ],
    out_specs=[
        pl.BlockSpec((Q, TK), lambda b: (0, 0)),
        pl.BlockSpec((Q, TK), lambda b: (0, 0)),
    ],
    out_shape=[
        jax.ShapeDtypeStruct((Q, TK), jnp.float32),
        jax.ShapeDtypeStruct((Q, TK), jnp.int32),
    ],
    scratch_shapes=[
        pltpu.VMEM((Q, D), jnp.bfloat16),
        pltpu.VMEM((Q, TK), jnp.float32),
        pltpu.VMEM((Q, TK), jnp.int32),
    ],
    compiler_params=pltpu.CompilerParams(dimension_semantics=("arbitrary",)),
)


def _gather_body(idx_hbm, tab_hbm, out_hbm, idx_v, rows_v, sem):
    wid = lax.axis_index("s") * _NC + lax.axis_index("c")
    base = wid * _RPW
    pltpu.sync_copy(idx_hbm.at[pl.ds(base, _RPW)], idx_v)
    c0 = pltpu.async_copy(
        tab_hbm.at[idx_v.at[pl.ds(0, _CH)]], rows_v.at[pl.ds(0, _CH)], sem)
    c1 = pltpu.async_copy(
        tab_hbm.at[idx_v.at[pl.ds(_CH, _CH)]], rows_v.at[pl.ds(_CH, _CH)], sem)
    c0.wait()
    c1.wait()
    pltpu.sync_copy(rows_v, out_hbm.at[pl.ds(base, _RPW)])


@functools.cache
def _gather_call():
    return pl.kernel(
        _gather_body,
        out_type=jax.ShapeDtypeStruct((Q * TK, D), jnp.float32),
        mesh=plsc.VectorSubcoreMesh(core_axis_name="c", subcore_axis_name="s"),
        scratch_types=[
            pltpu.VMEM((_RPW,), jnp.int32),
            pltpu.VMEM((_RPW, D), jnp.float32),
            pltpu.SemaphoreType.DMA,
        ],
    )


def _evnorm_body(e_ref, o_ref):
    e = e_ref[...]
    sq = jnp.sum(e * e, axis=1, keepdims=True)
    o_ref[...] = e / (jnp.sqrt(sq) + 1e-12)


_evnorm_call = pl.pallas_call(
    _evnorm_body,
    grid=(8,),
    in_specs=[pl.BlockSpec((Q * TK // 8, D), lambda i: (i, 0))],
    out_specs=pl.BlockSpec((Q * TK // 8, D), lambda i: (i, 0)),
    out_shape=jax.ShapeDtypeStruct((Q * TK, D), jnp.float32),
)


def kernel(queries, keys):
    top_vals, top_idx = _topk_call(queries, keys)
    flat_idx = top_idx.reshape(Q * TK)
    ev_raw = _gather_call()(flat_idx, keys)
    evidence = _evnorm_call(ev_raw).reshape(Q, TK, D)
    return top_vals, top_idx, evidence


# speedup vs baseline: 2.7805x; 1.2187x over previous
import functools

import jax
import jax.numpy as jnp
from jax import lax
from jax.experimental import pallas as pl
from jax.experimental.pallas import tpu as pltpu
from jax.experimental.pallas import tpu_sc as plsc

Q = 1024
D = 768
K = 100000
TK = 5
KB = 2048
NB = (K + KB - 1) // KB

_NC = 2
_NS = 16
_NW = _NC * _NS
_RPW = (Q * TK) // _NW
_CH = _RPW // 2


def _topk_body(q_ref, k_ref, vals_ref, idx_ref, qbf_ref, *rr):
    b = pl.program_id(0)

    @pl.when(b == 0)
    def _init():
        q = q_ref[...]
        qs = jnp.sum(q * q, axis=1, keepdims=True)
        qn = q * (1.0 / (jnp.sqrt(qs) + 1e-12))
        qbf_ref[...] = qn.astype(jnp.bfloat16)
        for j in range(TK):
            rr[j][...] = jnp.full((1, Q), -jnp.inf, jnp.float32)
            rr[TK + j][...] = jnp.zeros((1, Q), jnp.int32)

    kb = k_ref[...]
    ksq = jnp.sum(kb * kb, axis=1, keepdims=True)
    kinv = 1.0 / (jnp.sqrt(ksq) + 1e-12)
    knb = (kb * kinv).astype(jnp.bfloat16)
    s = lax.dot_general(
        knb, qbf_ref[...], (((1,), (1,)), ((), ())),
        preferred_element_type=jnp.float32,
    )
    base = b * KB
    rowf = lax.broadcasted_iota(jnp.int32, (KB, Q), 0).astype(jnp.float32)
    s = jnp.where(rowf < (K - base).astype(jnp.float32), s, -jnp.inf)

    rv = [rr[j][...] for j in range(TK)]
    ri = [rr[TK + j][...] for j in range(TK)]
    for t in range(TK):
        m = jnp.max(s, axis=0, keepdims=True)
        eq = s == m
        amf = jnp.min(jnp.where(eq, rowf, jnp.inf), axis=0, keepdims=True)
        if t < TK - 1:
            s = jnp.where(rowf == amf, -jnp.inf, s)
        gi = amf.astype(jnp.int32) + base
        xv, xi = m, gi
        for j in range(TK):
            sw = (xv > rv[j]) | ((xv == rv[j]) & (xi < ri[j]))
            rv[j], xv = jnp.where(sw, xv, rv[j]), jnp.where(sw, rv[j], xv)
            ri[j], xi = jnp.where(sw, xi, ri[j]), jnp.where(sw, ri[j], xi)
    for j in range(TK):
        rr[j][...] = rv[j]
        rr[TK + j][...] = ri[j]

    @pl.when(b == NB - 1)
    def _fin():
        for j in range(TK):
            vals_ref[j : j + 1, :] = rv[j]
            idx_ref[j : j + 1, :] = ri[j]


_topk_call = pl.pallas_call(
    _topk_body,
    grid=(NB,),
    in_specs=[
        pl.BlockSpec((Q, D), lambda b: (0, 0)),
        pl.BlockSpec((KB, D), lambda b: (b, 0)),
    ],
    out_specs=[
        pl.BlockSpec((TK, Q), lambda b: (0, 0)),
        pl.BlockSpec((TK, Q), lambda b: (0, 0)),
    ],
    out_shape=[
        jax.ShapeDtypeStruct((TK, Q), jnp.float32),
        jax.ShapeDtypeStruct((TK, Q), jnp.int32),
    ],
    scratch_shapes=(
        [pltpu.VMEM((Q, D), jnp.bfloat16)]
        + [pltpu.VMEM((1, Q), jnp.float32) for _ in range(TK)]
        + [pltpu.VMEM((1, Q), jnp.int32) for _ in range(TK)]
    ),
    compiler_params=pltpu.CompilerParams(dimension_semantics=("arbitrary",)),
)


def _gather_body(idx_hbm, tab_hbm, out_hbm, idx_v, rows_v, sem):
    wid = lax.axis_index("s") * _NC + lax.axis_index("c")
    base = wid * _RPW
    pltpu.sync_copy(idx_hbm.at[pl.ds(base, _RPW)], idx_v)
    c0 = pltpu.async_copy(
        tab_hbm.at[idx_v.at[pl.ds(0, _CH)]], rows_v.at[pl.ds(0, _CH)], sem)
    c1 = pltpu.async_copy(
        tab_hbm.at[idx_v.at[pl.ds(_CH, _CH)]], rows_v.at[pl.ds(_CH, _CH)], sem)
    c0.wait()
    c1.wait()
    pltpu.sync_copy(rows_v, out_hbm.at[pl.ds(base, _RPW)])


@functools.cache
def _gather_call():
    return pl.kernel(
        _gather_body,
        out_type=jax.ShapeDtypeStruct((Q * TK, D), jnp.float32),
        mesh=plsc.VectorSubcoreMesh(core_axis_name="c", subcore_axis_name="s"),
        scratch_types=[
            pltpu.VMEM((_RPW,), jnp.int32),
            pltpu.VMEM((_RPW, D), jnp.float32),
            pltpu.SemaphoreType.DMA,
        ],
    )


def _evnorm_body(e_ref, o_ref):
    e = e_ref[...]
    sq = jnp.sum(e * e, axis=1, keepdims=True)
    o_ref[...] = e / (jnp.sqrt(sq) + 1e-12)


_evnorm_call = pl.pallas_call(
    _evnorm_body,
    grid=(8,),
    in_specs=[pl.BlockSpec((Q * TK // 8, D), lambda i: (i, 0))],
    out_specs=pl.BlockSpec((Q * TK // 8, D), lambda i: (i, 0)),
    out_shape=jax.ShapeDtypeStruct((Q * TK, D), jnp.float32),
)


def kernel(queries, keys):
    tv_t, ti_t = _topk_call(queries, keys)
    top_vals, top_idx = tv_t.T, ti_t.T
    flat_idx = top_idx.reshape(Q * TK)
    ev_raw = _gather_call()(flat_idx, keys)
    evidence = _evnorm_call(ev_raw).reshape(Q, TK, D)
    return top_vals, top_idx, evidence
